# Initial kernel scaffold; baseline (speedup 1.0000x reference)
import jax
import jax.numpy as jnp
from jax.experimental import pallas as pl


def kernel(input_ids, word_embeddings, position_embeddings, ln_scale, ln_bias):
    raise NotImplementedError("write your pallas kernel here")


# speedup vs baseline: 2.6941x; 2.6941x over previous
import functools

import jax
import jax.numpy as jnp
from jax import lax
from jax.experimental import pallas as pl
from jax.experimental.pallas import tpu as pltpu
from jax.experimental.pallas import tpu_sc as plsc

NC = 2
NS = 16
NW = NC * NS
LANES = 16
EPS = 1e-5
CHUNK = 128


def _xlane_sum(x):
    lanes = lax.iota(jnp.int32, 16)
    for sh in (8, 4, 2, 1):
        x = x + x.at[lanes ^ sh].get(mode="promise_in_bounds")
    return x


def _rsqrt(v):
    bits = lax.bitcast_convert_type(v, jnp.int32)
    seed = lax.bitcast_convert_type(
        jnp.int32(0x5F3759DF) - lax.shift_right_logical(bits, 1), jnp.float32)
    y = seed
    for _ in range(3):
        y = y * (1.5 - 0.5 * v * y * y)
    return y


def _sc_embed_ln(ids_flat, table, pos, scale, bias, *, n_rows, hidden, seq):
    rows_per_w = n_rows // NW
    n_chunks = rows_per_w // CHUNK
    nvec = hidden // LANES
    mesh = plsc.VectorSubcoreMesh(
        core_axis_name="c", subcore_axis_name="s",
        num_cores=NC, num_subcores=NS)

    @functools.partial(
        pl.kernel,
        out_type=jax.ShapeDtypeStruct((n_rows, hidden), jnp.float32),
        mesh=mesh,
        scratch_types=[
            pltpu.VMEM((seq, hidden), jnp.float32),
            pltpu.VMEM((hidden,), jnp.float32),
            pltpu.VMEM((hidden,), jnp.float32),
            pltpu.VMEM((2, CHUNK), jnp.int32),
            pltpu.VMEM((2, CHUNK, hidden), jnp.float32),
            pltpu.SemaphoreType.DMA,
            pltpu.SemaphoreType.DMA,
        ],
    )
    def kern(ids_hbm, table_hbm, pos_hbm, scale_hbm, bias_hbm, out_hbm,
             pos_v, scale_v, bias_v, idx_v, rows_v, sem0, sem1):
        wid = lax.axis_index("s") * NC + lax.axis_index("c")
        base0 = wid * rows_per_w

        pltpu.sync_copy(pos_hbm, pos_v)
        pltpu.sync_copy(scale_hbm, scale_v)
        pltpu.sync_copy(bias_hbm, bias_v)

        sems = (sem0, sem1)

        def start_gather(c, b):
            pltpu.sync_copy(ids_hbm.at[pl.ds(base0 + c * CHUNK, CHUNK)],
                            idx_v.at[b])
            pltpu.async_copy(table_hbm.at[idx_v.at[b]], rows_v.at[b], sems[b])

        def wait_gather(b):
            pltpu.make_async_copy(table_hbm.at[idx_v.at[b]], rows_v.at[b],
                                  sems[b]).wait()

        def compute_chunk(c, b):
            base = base0 + c * CHUNK
            buf = rows_v.at[b]

            @plsc.parallel_loop(0, CHUNK, 1, unroll=2)
            def _(j):
                s = lax.rem(base + j, seq)
                x = [buf[j, pl.ds(k * LANES, LANES)]
                     + pos_v[s, pl.ds(k * LANES, LANES)]
                     for k in range(nvec)]
                acc = x[0]
                for k in range(1, nvec):
                    acc = acc + x[k]
                mean = _xlane_sum(acc) * (1.0 / hidden)
                d = [xk - mean for xk in x]
                vacc = d[0] * d[0]
                for k in range(1, nvec):
                    vacc = vacc + d[k] * d[k]
                var = _xlane_sum(vacc) * (1.0 / hidden)
                inv = _rsqrt(var + EPS)
                for k in range(nvec):
                    sl = pl.ds(k * LANES, LANES)
                    buf[j, sl] = d[k] * inv * scale_v[sl] + bias_v[sl]

            pltpu.sync_copy(buf, out_hbm.at[pl.ds(base, CHUNK)])

        start_gather(0, 0)

        @pl.loop(0, n_chunks // 2)
        def _(g):
            c0 = g * 2
            for b in range(2):
                c = c0 + b
                nb = 1 - b

                @pl.when(c + 1 < n_chunks)
                def _():
                    start_gather(c + 1, nb)

                wait_gather(b)
                compute_chunk(c, b)

    return kern(ids_flat, table, pos, scale, bias)


def kernel(input_ids, word_embeddings, position_embeddings, ln_scale, ln_bias):
    batch, seq = input_ids.shape
    vocab, hidden = word_embeddings.shape
    n_rows = batch * seq
    ids_flat = input_ids.reshape(n_rows).astype(jnp.int32)
    out = _sc_embed_ln(
        ids_flat, word_embeddings,
        position_embeddings[:seq].astype(jnp.float32),
        ln_scale, ln_bias,
        n_rows=n_rows, hidden=hidden, seq=seq)
    return out.reshape(batch, seq, hidden)


# speedup vs baseline: 4.5834x; 1.7013x over previous
import functools

import jax
import jax.numpy as jnp
from jax import lax
from jax.experimental import pallas as pl
from jax.experimental.pallas import tpu as pltpu
from jax.experimental.pallas import tpu_sc as plsc

NC = 2
NS = 16
NW = NC * NS
LANES = 16
EPS = 1e-5
CHUNK = 128
NBUF = 5
LEAD = 2


def _xlane_sum(x):
    lanes = lax.iota(jnp.int32, 16)
    for sh in (8, 4, 2, 1):
        x = x + x.at[lanes ^ sh].get(mode="promise_in_bounds")
    return x


def _rsqrt(v):
    bits = lax.bitcast_convert_type(v, jnp.int32)
    y = lax.bitcast_convert_type(
        jnp.int32(0x5F3759DF) - lax.shift_right_logical(bits, 1), jnp.float32)
    h = 0.5 * v
    for _ in range(2):
        y = y * (1.5 - h * y * y)
    return y


def _sc_embed_ln(ids_2d, table, pos, scale, bias, *, n_rows, hidden, seq):
    rows_per_w = n_rows // NW
    n_chunks = rows_per_w // CHUNK
    nvec = hidden // LANES
    mesh = plsc.VectorSubcoreMesh(
        core_axis_name="c", subcore_axis_name="s",
        num_cores=NC, num_subcores=NS)

    @functools.partial(
        pl.kernel,
        out_type=jax.ShapeDtypeStruct((n_rows, hidden), jnp.float32),
        mesh=mesh,
        scratch_types=[
            pltpu.VMEM((seq, hidden), jnp.float32),
            pltpu.VMEM((hidden,), jnp.float32),
            pltpu.VMEM((hidden,), jnp.float32),
            pltpu.VMEM((n_chunks, CHUNK), jnp.int32),
            pltpu.VMEM((NBUF, CHUNK, hidden), jnp.float32),
            [pltpu.SemaphoreType.DMA] * NBUF,
            [pltpu.SemaphoreType.DMA] * NBUF,
        ],
    )
    def kern(ids_hbm, table_hbm, pos_hbm, scale_hbm, bias_hbm, out_hbm,
             pos_v, scale_v, bias_v, idx_v, rows_v, gsems, osems):
        wid = lax.axis_index("s") * NC + lax.axis_index("c")
        base0 = wid * rows_per_w

        pltpu.sync_copy(ids_hbm.at[wid], idx_v)
        pltpu.sync_copy(pos_hbm, pos_v)
        pltpu.sync_copy(scale_hbm, scale_v)
        pltpu.sync_copy(bias_hbm, bias_v)

        def gather(c, b):
            return pltpu.make_async_copy(
                table_hbm.at[idx_v.at[c]], rows_v.at[b], gsems[b])

        def out_write(c, b):
            return pltpu.make_async_copy(
                rows_v.at[b], out_hbm.at[pl.ds(base0 + c * CHUNK, CHUNK)],
                osems[b])

        def compute_chunk(c, b):
            base = base0 + c * CHUNK
            buf = rows_v.at[b]
            sc = [scale_v[pl.ds(k * LANES, LANES)] for k in range(nvec)]
            bi = [bias_v[pl.ds(k * LANES, LANES)] for k in range(nvec)]

            @plsc.parallel_loop(0, CHUNK, 1, unroll=4)
            def _(j):
                s = lax.rem(base + j, seq)
                x = [buf[j, pl.ds(k * LANES, LANES)]
                     + pos_v[s, pl.ds(k * LANES, LANES)]
                     for k in range(nvec)]
                acc = x[0]
                for k in range(1, nvec):
                    acc = acc + x[k]
                m = _xlane_sum(acc) * (1.0 / hidden)
                d = [xk - m for xk in x]
                vacc = d[0] * d[0]
                for k in range(1, nvec):
                    vacc = vacc + d[k] * d[k]
                r = _rsqrt(_xlane_sum(vacc) * (1.0 / hidden) + EPS)
                for k in range(nvec):
                    buf[j, pl.ds(k * LANES, LANES)] = (d[k] * r) * sc[k] + bi[k]

        for c in range(LEAD):
            gather(c, c).start()

        @pl.loop(0, n_chunks // NBUF)
        def _(g):
            c0 = g * NBUF
            for b in range(NBUF):
                c = c0 + b
                gather(c, b).wait()
                compute_chunk(c, b)
                out_write(c, b).start()

                nc = c + LEAD
                nb = (b + LEAD) % NBUF

                @pl.when(nc < n_chunks)
                def _():
                    @pl.when(c >= NBUF - LEAD)
                    def _():
                        out_write(nc - NBUF, nb).wait()

                    gather(nc, nb).start()

        for t in range(NBUF):
            out_write(n_chunks - NBUF + t, (n_chunks - NBUF + t) % NBUF).wait()

    return kern(ids_2d, table, pos, scale, bias)


def kernel(input_ids, word_embeddings, position_embeddings, ln_scale, ln_bias):
    batch, seq = input_ids.shape
    vocab, hidden = word_embeddings.shape
    n_rows = batch * seq
    ids_2d = input_ids.reshape(
        NW, n_rows // (NW * CHUNK), CHUNK).astype(jnp.int32)
    out = _sc_embed_ln(
        ids_2d, word_embeddings,
        position_embeddings[:seq].astype(jnp.float32),
        ln_scale, ln_bias,
        n_rows=n_rows, hidden=hidden, seq=seq)
    return out.reshape(batch, seq, hidden)
